# baseline (speedup 1.0000x reference)
import functools

import jax
import jax.numpy as jnp
from jax import lax
from jax.experimental import pallas as pl
from jax.experimental.pallas import tpu as pltpu
from jax.experimental.pallas import tpu_sc as plsc

D = 64
NC, NS = 2, 16
NW = NC * NS
TOTAL = 4096 * 200
PER_W = TOTAL // NW
C = 128
ITERS = PER_W // C

_mesh = plsc.VectorSubcoreMesh(core_axis_name="c", subcore_axis_name="s")


@functools.partial(
    pl.kernel,
    out_type=jax.ShapeDtypeStruct((TOTAL, D), jnp.float32),
    mesh=_mesh,
    scratch_types=[
        pltpu.VMEM((PER_W,), jnp.int32),
        pltpu.VMEM((C, D), jnp.float32),
        pltpu.SemaphoreType.DMA,
    ],
    compiler_params=pltpu.CompilerParams(use_tc_tiling_on_sc=False),
)
def _gather_kernel(table_hbm, idx_hbm, out_hbm, idx_v, rows_v, sem):
    wid = lax.axis_index("s") * NC + lax.axis_index("c")
    base = wid * PER_W
    pltpu.sync_copy(idx_hbm.at[pl.ds(base, PER_W)], idx_v)

    @pl.loop(0, ITERS)
    def _(i):
        off = i * C
        pltpu.async_copy(table_hbm.at[idx_v.at[pl.ds(off, C)]], rows_v, sem).wait()
        pltpu.sync_copy(rows_v, out_hbm.at[pl.ds(base + off, C)])


def kernel(indices, table):
    flat = indices.reshape(-1).astype(jnp.int32)
    out = _gather_kernel(table, flat)
    return out.reshape(indices.shape[0], indices.shape[1], D)


# speedup vs baseline: 1.0867x; 1.0867x over previous
import functools

import jax
import jax.numpy as jnp
from jax import lax
from jax.experimental import pallas as pl
from jax.experimental.pallas import tpu as pltpu
from jax.experimental.pallas import tpu_sc as plsc

D = 64
NC, NS = 2, 16
NW = NC * NS
TOTAL = 4096 * 200
PER_W = TOTAL // NW
C = 128
ITERS = PER_W // C

_mesh = plsc.VectorSubcoreMesh(core_axis_name="c", subcore_axis_name="s")


@functools.partial(
    pl.kernel,
    out_type=jax.ShapeDtypeStruct((TOTAL, D), jnp.float32),
    mesh=_mesh,
    scratch_types=[
        pltpu.VMEM((PER_W,), jnp.int32),
        pltpu.VMEM((C * 4, D), jnp.float32),
        pltpu.SemaphoreType.DMA,
    ],
    compiler_params=pltpu.CompilerParams(use_tc_tiling_on_sc=False),
)
def _gather_kernel(table_hbm, idx_hbm, out_hbm, idx_v, rows_v, sem):
    wid = lax.axis_index("s") * NC + lax.axis_index("c")
    base = wid * PER_W
    pltpu.sync_copy(idx_hbm.at[pl.ds(base, PER_W)], idx_v)

    @pl.loop(0, ITERS // 4)
    def _(i):
        off = i * C * 4
        src = table_hbm.at[idx_v.at[pl.ds(off, C * 4)]]
        pltpu.async_copy(src, rows_v, sem).wait()
        pltpu.sync_copy(rows_v, out_hbm.at[pl.ds(base + off, C * 4)])


def kernel(indices, table):
    flat = indices.reshape(-1).astype(jnp.int32)
    out = _gather_kernel(table, flat)
    return out.reshape(indices.shape[0], indices.shape[1], D)


# speedup vs baseline: 1.1133x; 1.0245x over previous
import functools

import jax
import jax.numpy as jnp
from jax import lax
from jax.experimental import pallas as pl
from jax.experimental.pallas import tpu as pltpu
from jax.experimental.pallas import tpu_sc as plsc

D = 64
NC, NS = 2, 16
NW = NC * NS
TOTAL = 4096 * 200
PER_W = TOTAL // NW
G = 512
NG = PER_W // G

_mesh = plsc.VectorSubcoreMesh(core_axis_name="c", subcore_axis_name="s")


@functools.partial(
    pl.kernel,
    out_type=jax.ShapeDtypeStruct((TOTAL, D), jnp.float32),
    mesh=_mesh,
    scratch_types=[
        pltpu.VMEM((PER_W,), jnp.int32),
        pltpu.VMEM((G, D), jnp.float32),
        pltpu.VMEM((G, D), jnp.float32),
        pltpu.SemaphoreType.DMA,
        pltpu.SemaphoreType.DMA,
        pltpu.SemaphoreType.DMA,
        pltpu.SemaphoreType.DMA,
    ],
    compiler_params=pltpu.CompilerParams(use_tc_tiling_on_sc=False),
)
def _gather_kernel(table_hbm, idx_hbm, out_hbm, idx_v, ra, rb, gsa, gsb, osa, osb):
    wid = lax.axis_index("s") * NC + lax.axis_index("c")
    base = wid * PER_W
    pltpu.sync_copy(idx_hbm.at[pl.ds(base, PER_W)], idx_v)

    def fire_g(group, buf, gsem):
        pltpu.async_copy(table_hbm.at[idx_v.at[pl.ds(group * G, G)]], buf, gsem)

    def drain_g(buf, gsem):
        pltpu.make_async_copy(table_hbm.at[idx_v.at[pl.ds(0, G)]], buf, gsem).wait()

    def fire_w(group, buf, osem):
        pltpu.async_copy(buf, out_hbm.at[pl.ds(base + group * G, G)], osem)

    def drain_w(buf, osem):
        pltpu.make_async_copy(buf, out_hbm.at[pl.ds(base, G)], osem).wait()

    fire_g(0, ra, gsa)
    fire_g(1, rb, gsb)
    drain_g(ra, gsa)
    fire_w(0, ra, osa)

    @pl.loop(0, (NG - 2) // 2)
    def _(t):
        g = 2 * t + 1
        drain_w(ra, osa)
        fire_g(g + 1, ra, gsa)
        drain_g(rb, gsb)
        fire_w(g, rb, osb)
        g2 = g + 1
        drain_w(rb, osb)
        fire_g(g2 + 1, rb, gsb)
        drain_g(ra, gsa)
        fire_w(g2, ra, osa)

    drain_g(rb, gsb)
    fire_w(NG - 1, rb, osb)
    drain_w(ra, osa)
    drain_w(rb, osb)


def kernel(indices, table):
    flat = indices.reshape(-1).astype(jnp.int32)
    out = _gather_kernel(table, flat)
    return out.reshape(indices.shape[0], indices.shape[1], D)
